# baseline (speedup 1.0000x reference)
import functools
import math

import jax
import jax.numpy as jnp
from jax.experimental import pallas as pl
from jax.experimental.pallas import tpu as pltpu

N = 116
NP = 128
HID = [32, 64, 128, 256, 512]
HEADS = [8, 4, 2, 1]
NEG = -1e9


def _gt_layer(h, mask, Wq, Wk, Wv, Wr, b, heads):
    d_out = Wq.shape[1]
    hd = d_out // heads
    scale = 1.0 / math.sqrt(hd)
    q = jnp.dot(h, Wq, preferred_element_type=jnp.float32) * scale
    k = jnp.dot(h, Wk, preferred_element_type=jnp.float32)
    v = jnp.dot(h, Wv, preferred_element_type=jnp.float32)
    outs = []
    for hh in range(heads):
        qs = q[:, hh * hd:(hh + 1) * hd]
        ks = k[:, hh * hd:(hh + 1) * hd]
        vs = v[:, hh * hd:(hh + 1) * hd]
        logits = jax.lax.dot_general(
            qs, ks, (((1,), (1,)), ((), ())),
            preferred_element_type=jnp.float32)
        logits = jnp.where(mask, logits, NEG)
        m = jnp.max(logits, axis=1, keepdims=True)
        e = jnp.where(mask, jnp.exp(logits - m), 0.0)
        s = jnp.sum(e, axis=1, keepdims=True)
        alpha = e / jnp.maximum(s, 1e-30)
        outs.append(jnp.dot(alpha, vs, preferred_element_type=jnp.float32))
    out = jnp.concatenate(outs, axis=1)
    r = jnp.dot(h, Wr, preferred_element_type=jnp.float32)
    return jnp.maximum(out + r + b, 0.0)


def _fwd_kernel(x_ref, adjT_ref, W_in_ref, b_in_ref,
                Wq1, Wk1, Wv1, Wr1, b1,
                Wq2, Wk2, Wv2, Wr2, b2,
                Wq3, Wk3, Wv3, Wr3, b3,
                Wq4, Wk4, Wv4, Wr4, b4,
                Wa_ref, Wfh_ref, Wfa_ref, bf_ref,
                att_ref, logit_ref):
    x = x_ref[0]
    mask = adjT_ref[0] > 0.0

    h = jnp.dot(x, W_in_ref[...], preferred_element_type=jnp.float32) + b_in_ref[...]
    h = _gt_layer(h, mask, Wq1[...], Wk1[...], Wv1[...], Wr1[...], b1[...], 8)
    h = _gt_layer(h, mask, Wq2[...], Wk2[...], Wv2[...], Wr2[...], b2[...], 4)
    h = _gt_layer(h, mask, Wq3[...], Wk3[...], Wv3[...], Wr3[...], b3[...], 2)
    h = _gt_layer(h, mask, Wq4[...], Wk4[...], Wv4[...], Wr4[...], b4[...], 1)

    hw = jnp.dot(h, Wa_ref[...], preferred_element_type=jnp.float32)
    scores = jax.lax.dot_general(
        hw, h, (((1,), (1,)), ((), ())),
        preferred_element_type=jnp.float32) * (1.0 / math.sqrt(HID[4]))
    colv = jax.lax.broadcasted_iota(jnp.int32, (NP, NP), 1) < N
    scores = jnp.where(colv, scores, NEG)
    m = jnp.max(scores, axis=1, keepdims=True)
    e = jnp.where(colv, jnp.exp(scores - m), 0.0)
    att = e / jnp.sum(e, axis=1, keepdims=True)
    att_ref[0] = att

    rowv = jax.lax.broadcasted_iota(jnp.int32, (NP, 1), 0) < N
    inv_n = 1.0 / N
    pooled_h = jnp.sum(jnp.where(rowv, h, 0.0), axis=0, keepdims=True) * inv_n
    pooled_a = jnp.sum(jnp.where(rowv, att, 0.0), axis=0, keepdims=True) * inv_n
    logit = (jnp.dot(pooled_h, Wfh_ref[...], preferred_element_type=jnp.float32)
             + jnp.dot(pooled_a, Wfa_ref[...], preferred_element_type=jnp.float32)
             + bf_ref[...])
    logit_ref[0] = logit


def kernel(x, adj, W_in, b_in, Wq1, Wk1, Wv1, Wr1, b1, Wq2, Wk2, Wv2, Wr2, b2,
           Wq3, Wk3, Wv3, Wr3, b3, Wq4, Wk4, Wv4, Wr4, b4, Wa, Wf, bf):
    B = x.shape[0]
    f32 = jnp.float32

    xp = jnp.pad(x, ((0, 0), (0, NP - N), (0, NP - N)))
    adjT = jnp.pad(jnp.swapaxes(adj, 1, 2), ((0, 0), (0, NP - N), (0, NP - N)))
    W_in_p = jnp.pad(W_in, ((0, NP - N), (0, 0)))
    Wfh = Wf[:HID[4]]
    Wfa = jnp.pad(Wf[HID[4]:], ((0, NP - N), (0, 0)))
    b_in2 = b_in.reshape(1, -1)
    bs = [b1.reshape(1, -1), b2.reshape(1, -1), b3.reshape(1, -1), b4.reshape(1, -1)]
    bf2 = bf.reshape(1, -1)

    def wspec(a):
        return pl.BlockSpec(a.shape, lambda b: (0,) * a.ndim)

    layer_ws = [Wq1, Wk1, Wv1, Wr1, bs[0],
                Wq2, Wk2, Wv2, Wr2, bs[1],
                Wq3, Wk3, Wv3, Wr3, bs[2],
                Wq4, Wk4, Wv4, Wr4, bs[3]]

    in_specs = [
        pl.BlockSpec((1, NP, NP), lambda b: (b, 0, 0)),
        pl.BlockSpec((1, NP, NP), lambda b: (b, 0, 0)),
        wspec(W_in_p), wspec(b_in2),
    ] + [wspec(w) for w in layer_ws] + [
        wspec(Wa), wspec(Wfh), wspec(Wfa), wspec(bf2),
    ]

    out_shapes = (
        jax.ShapeDtypeStruct((B, NP, NP), f32),
        jax.ShapeDtypeStruct((B, 1, 2), f32),
    )
    out_specs = (
        pl.BlockSpec((1, NP, NP), lambda b: (b, 0, 0)),
        pl.BlockSpec((1, 1, 2), lambda b: (b, 0, 0)),
    )

    att_p, logit3 = pl.pallas_call(
        _fwd_kernel,
        grid=(B,),
        in_specs=in_specs,
        out_specs=out_specs,
        out_shape=out_shapes,
        compiler_params=pltpu.CompilerParams(
            dimension_semantics=("parallel",)),
    )(xp, adjT, W_in_p, b_in2, *layer_ws, Wa, Wfh, Wfa, bf2)

    attention = att_p[:, :N, :N]
    logit = logit3[:, 0, :]
    return (attention, logit)


# speedup vs baseline: 1.9636x; 1.9636x over previous
import math

import jax
import jax.numpy as jnp
from jax.experimental import pallas as pl
from jax.experimental.pallas import tpu as pltpu

N = 116
NP = 128
G = 8
HID = [32, 64, 128, 256, 512]
HEADS = [8, 4, 2, 1]
NEG = -1e9

_f32 = jnp.float32


def _dot(a, b):
    return jnp.dot(a, b, preferred_element_type=_f32)


def _dot_t(a, b):
    return jax.lax.dot_general(a, b, (((1,), (1,)), ((), ())),
                               preferred_element_type=_f32)


def _gt_layer(h, penalties, rowhas, ones, Wq, Wk, Wv, Wr, b, heads):
    d_out = Wq.shape[1]
    hd = d_out // heads
    scale = 1.0 / math.sqrt(hd)
    q = _dot(h, Wq) * scale
    k = _dot(h, Wk)
    v = _dot(h, Wv)
    r = _dot(h, Wr)
    outs = []
    for g in range(G):
        sl = slice(g * NP, (g + 1) * NP)
        qg, kg, vg = q[sl], k[sl], v[sl]
        head_outs = []
        for hh in range(heads):
            hsl = slice(hh * hd, (hh + 1) * hd)
            logits = _dot_t(qg[:, hsl], kg[:, hsl]) + penalties[g]
            m = jnp.max(logits, axis=1, keepdims=True)
            e = jnp.exp(logits - m)
            s = _dot(e, ones)
            o = _dot(e, vg[:, hsl])
            head_outs.append(o * (rowhas[g] / s))
        outs.append(jnp.concatenate(head_outs, axis=1))
    out = jnp.concatenate(outs, axis=0)
    return jnp.maximum(out + r + b, 0.0)


def _fwd_kernel(x_ref, adjT_ref, W_in_ref, b_in_ref,
                Wq1, Wk1, Wv1, Wr1, b1,
                Wq2, Wk2, Wv2, Wr2, b2,
                Wq3, Wk3, Wv3, Wr3, b3,
                Wq4, Wk4, Wv4, Wr4, b4,
                Wa_ref, Wfh_ref, Wfa_ref, bf_ref,
                att_ref, logit_ref):
    x = x_ref[...].reshape(G * NP, NP)
    ones = jnp.ones((NP, 1), _f32)

    penalties, rowhas = [], []
    for g in range(G):
        mf = (adjT_ref[g] > 0.0).astype(_f32)
        penalties.append((mf - 1.0) * 1e9)
        rowhas.append((_dot(mf, ones) > 0.0).astype(_f32))

    h = _dot(x, W_in_ref[...]) + b_in_ref[...]
    h = _gt_layer(h, penalties, rowhas, ones, Wq1[...], Wk1[...], Wv1[...], Wr1[...], b1[...], 8)
    h = _gt_layer(h, penalties, rowhas, ones, Wq2[...], Wk2[...], Wv2[...], Wr2[...], b2[...], 4)
    h = _gt_layer(h, penalties, rowhas, ones, Wq3[...], Wk3[...], Wv3[...], Wr3[...], b3[...], 2)
    h = _gt_layer(h, penalties, rowhas, ones, Wq4[...], Wk4[...], Wv4[...], Wr4[...], b4[...], 1)

    hw = _dot(h, Wa_ref[...])
    colpen = jnp.where(
        jax.lax.broadcasted_iota(jnp.int32, (NP, NP), 1) < N, 0.0, NEG)
    rowv = jnp.where(
        jax.lax.broadcasted_iota(jnp.int32, (1, NP), 1) < N, 1.0 / N, 0.0)
    fscale = 1.0 / math.sqrt(HID[4])
    for g in range(G):
        sl = slice(g * NP, (g + 1) * NP)
        hg = h[sl]
        scores = _dot_t(hw[sl], hg) * fscale + colpen
        m = jnp.max(scores, axis=1, keepdims=True)
        e = jnp.exp(scores - m)
        s = _dot(e, ones)
        att = e / s
        att_ref[g] = att
        pooled_h = _dot(rowv, hg)
        pooled_a = _dot(rowv, att)
        logit_ref[g] = (_dot(pooled_h, Wfh_ref[...])
                        + _dot(pooled_a, Wfa_ref[...]) + bf_ref[...])


def kernel(x, adj, W_in, b_in, Wq1, Wk1, Wv1, Wr1, b1, Wq2, Wk2, Wv2, Wr2, b2,
           Wq3, Wk3, Wv3, Wr3, b3, Wq4, Wk4, Wv4, Wr4, b4, Wa, Wf, bf):
    B = x.shape[0]

    xp = jnp.pad(x, ((0, 0), (0, NP - N), (0, NP - N)))
    adjT = jnp.pad(jnp.swapaxes(adj, 1, 2), ((0, 0), (0, NP - N), (0, NP - N)))
    W_in_p = jnp.pad(W_in, ((0, NP - N), (0, 0)))
    Wfh = Wf[:HID[4]]
    Wfa = jnp.pad(Wf[HID[4]:], ((0, NP - N), (0, 0)))
    b_in2 = b_in.reshape(1, -1)
    bs = [b1.reshape(1, -1), b2.reshape(1, -1), b3.reshape(1, -1), b4.reshape(1, -1)]
    bf2 = bf.reshape(1, -1)

    def wspec(a):
        return pl.BlockSpec(a.shape, lambda b: (0,) * a.ndim)

    layer_ws = [Wq1, Wk1, Wv1, Wr1, bs[0],
                Wq2, Wk2, Wv2, Wr2, bs[1],
                Wq3, Wk3, Wv3, Wr3, bs[2],
                Wq4, Wk4, Wv4, Wr4, bs[3]]

    in_specs = [
        pl.BlockSpec((G, NP, NP), lambda b: (b, 0, 0)),
        pl.BlockSpec((G, NP, NP), lambda b: (b, 0, 0)),
        wspec(W_in_p), wspec(b_in2),
    ] + [wspec(w) for w in layer_ws] + [
        wspec(Wa), wspec(Wfh), wspec(Wfa), wspec(bf2),
    ]

    out_shapes = (
        jax.ShapeDtypeStruct((B, NP, NP), _f32),
        jax.ShapeDtypeStruct((B, 1, 2), _f32),
    )
    out_specs = (
        pl.BlockSpec((G, NP, NP), lambda b: (b, 0, 0)),
        pl.BlockSpec((G, 1, 2), lambda b: (b, 0, 0)),
    )

    att_p, logit3 = pl.pallas_call(
        _fwd_kernel,
        grid=(B // G,),
        in_specs=in_specs,
        out_specs=out_specs,
        out_shape=out_shapes,
        compiler_params=pltpu.CompilerParams(
            dimension_semantics=("parallel",)),
    )(xp, adjT, W_in_p, b_in2, *layer_ws, Wa, Wfh, Wfa, bf2)

    attention = att_p[:, :N, :N]
    logit = logit3[:, 0, :]
    return (attention, logit)
